# Initial kernel scaffold; baseline (speedup 1.0000x reference)
import jax
import jax.numpy as jnp
from jax.experimental import pallas as pl


def kernel(output, pos_output, neg_output, Y):
    raise NotImplementedError("write your pallas kernel here")


# speedup vs baseline: 3.1685x; 3.1685x over previous
import functools

import jax
import jax.numpy as jnp
from jax import lax
from jax.experimental import pallas as pl
from jax.experimental.pallas import tpu as pltpu

_B = 8192
_D = 128
_C = 100
_NBINS = 128
_BIN_LEN = 4.0
_EPS = 1e-07

_CHUNK = 1024
_GRID = _B // _CHUNK


def _row_labels(yblk):
    m = jnp.max(yblk, axis=1, keepdims=True)
    cols = lax.broadcasted_iota(jnp.int32, yblk.shape, 1)
    cand = jnp.where(yblk == m, cols, yblk.shape[1])
    return jnp.min(cand, axis=1, keepdims=True)


def _lane_cumsum(v):
    lane = lax.broadcasted_iota(jnp.int32, v.shape, 1)
    shift = 1
    while shift < v.shape[1]:
        rolled = pltpu.roll(v, shift, 1)
        v = v + jnp.where(lane >= shift, rolled, 0.0)
        shift *= 2
    return v


def _fastap_kernel(out_blk, y_blk, q_blk, y0_blk, loss_ref,
                   h_ref, hp_ref, np_ref):
    i = pl.program_id(0)

    @pl.when(i == 0)
    def _init():
        h_ref[...] = jnp.zeros_like(h_ref)
        hp_ref[...] = jnp.zeros_like(hp_ref)
        np_ref[0, 0] = 0.0

    labels = _row_labels(y_blk[...])
    label0 = _row_labels(y0_blk[...])
    eq = (labels == label0).astype(jnp.float32)

    np_ref[0, 0] += jnp.sum(eq)

    gidx = i * _CHUNK + lax.broadcasted_iota(jnp.int32, (_CHUNK, 1), 0)
    valid = (gidx >= 1).astype(jnp.float32)

    diff = out_blk[...] - q_blk[...]
    d = jnp.sum(diff * diff, axis=1, keepdims=True)

    centers = 2.0 + _BIN_LEN * lax.broadcasted_iota(
        jnp.int32, (_CHUNK, _NBINS), 1).astype(jnp.float32)
    delta = 1.0 - jnp.abs(centers - d) * (1.0 / _BIN_LEN)
    delta = jnp.maximum(delta, 0.0) * valid

    h_ref[...] += jnp.sum(delta, axis=0, keepdims=True)
    hp_ref[...] += jnp.sum(delta * eq, axis=0, keepdims=True)

    @pl.when(i == _GRID - 1)
    def _finish():
        h = h_ref[...]
        hp = hp_ref[...]
        big_h = _lane_cumsum(h)
        big_hp = _lane_cumsum(hp)
        loss = jnp.sum(hp * big_hp / (big_h + _EPS))
        loss_ref[0, 0] = loss / np_ref[0, 0]


@jax.jit
def _fastap(output, Y):
    q = output[0:1, :]
    y0 = Y[0:1, :]
    loss = pl.pallas_call(
        _fastap_kernel,
        grid=(_GRID,),
        in_specs=[
            pl.BlockSpec((_CHUNK, _D), lambda i: (i, 0)),
            pl.BlockSpec((_CHUNK, _C), lambda i: (i, 0)),
            pl.BlockSpec((1, _D), lambda i: (0, 0)),
            pl.BlockSpec((1, _C), lambda i: (0, 0)),
        ],
        out_specs=pl.BlockSpec(memory_space=pltpu.SMEM),
        out_shape=jax.ShapeDtypeStruct((1, 1), jnp.float32),
        scratch_shapes=[
            pltpu.VMEM((1, _NBINS), jnp.float32),
            pltpu.VMEM((1, _NBINS), jnp.float32),
            pltpu.SMEM((1, 1), jnp.float32),
        ],
    )(output, Y, q, y0)
    return loss[0, 0]


def kernel(output, pos_output, neg_output, Y):
    return _fastap(output, Y)
